# baseline (speedup 1.0000x reference)
import jax
import jax.numpy as jnp
from jax.experimental import pallas as pl
from jax.experimental.pallas import tpu as pltpu

_IN = 128
_BATCH = 1024
_NL = 5
_SHORT = 1000
_CUT = [1000, 10000, 40000, 100000]
_CS = [9000, 30000, 60000]
_HSZ = [64, 32, 16]
_HOFF = [0, 64, 96, 112]
_TILE = 1000
_TOFF = [0, 9, 39, 99]
_COFF = [0, 9000, 39000]
_NT = 99


_LOG2E = 1.4426950408889634
_LN2 = 0.6931471805599453


def _body(x_ref, hw_ref, w1_ref, gb_ref, targ_ref, w20_ref, w21_ref, w22_ref,
          out_ref, stats, hall, acc0, acc1, acc2):
    i = pl.program_id(0)
    accs = [acc0, acc1, acc2]

    @pl.when(i == 0)
    def _prep():
        x = x_ref[...]
        targ = targ_ref[:, 0:_NL]
        ho = jax.lax.dot_general(x, hw_ref[...], (((1,), (1,)), ((), ())),
                                 preferred_element_type=jnp.float32)
        hraw = jax.lax.dot_general(x, w1_ref[...], (((1,), (1,)), ((), ())),
                                   preferred_element_type=jnp.float32)
        g = gb_ref[0:1, :]
        b = gb_ref[1:2, :]
        for ci in range(3):
            lo, hi = _HOFF[ci], _HOFF[ci + 1]
            hseg = hraw[:, lo:hi]
            mu = jnp.mean(hseg, axis=1, keepdims=True)
            var = jnp.mean((hseg - mu) ** 2, axis=1, keepdims=True)
            hn = (hseg - mu) / jnp.sqrt(var + 1e-5) * g[:, lo:hi] + b[:, lo:hi]
            hall[:, lo:hi] = jnp.maximum(hn, 0.0) * (-_LOG2E)
        hall[:, _HOFF[3]:_IN] = jnp.zeros((_BATCH, _IN - _HOFF[3]), jnp.float32)

        colid = jax.lax.broadcasted_iota(jnp.int32, (_BATCH, 1024), 1)
        mh = jnp.zeros((_BATCH, 1024), jnp.bool_)
        for k in range(_NL):
            mh = mh | (targ[:, k:k + 1] == colid)
        mh = mh & (colid < _SHORT)
        sp = jnp.maximum(ho, 0.0) + jnp.log1p(jnp.exp(-jnp.abs(ho)))
        maskH = (colid < _SHORT).astype(jnp.float32)
        head_loss = jnp.sum(sp * maskH, axis=1, keepdims=True) \
            - jnp.sum(jnp.where(mh, ho, 0.0), axis=1, keepdims=True)

        num = jnp.full((_BATCH, 1), float(_SHORT), jnp.float32)
        for ci in range(3):
            lo, hi = _CUT[ci], _CUT[ci + 1]
            rm = jnp.zeros((_BATCH, 1), jnp.bool_)
            for k in range(_NL):
                tk = targ[:, k:k + 1]
                rm = rm | ((tk >= lo) & (tk < hi))
            rmf = rm.astype(jnp.float32)
            logit = ho[:, _SHORT + ci:_SHORT + ci + 1]
            sp_r = jnp.maximum(logit, 0.0) + jnp.log1p(jnp.exp(-jnp.abs(logit)))
            head_loss = head_loss + (1.0 - rmf) * sp_r
            num = num + jnp.where(rm, float(_CS[ci]), 1.0)
            stats[:, 2 + ci:3 + ci] = rmf
            stats[:, 5 + ci:6 + ci] = jax.nn.sigmoid(logit)
        stats[:, 0:1] = head_loss
        stats[:, 1:2] = num
        for k in range(_NL):
            tk = targ[:, k:k + 1]
            dup = jnp.zeros((_BATCH, 1), jnp.bool_)
            for j in range(k):
                dup = dup | (targ[:, j:j + 1] == tk)
            stats[:, 16 + k:17 + k] = jnp.where(
                dup, -1.0, (tk - _SHORT).astype(jnp.float32))
        for acc in (acc0, acc1, acc2):
            acc[...] = jnp.zeros((_BATCH, 128), jnp.float32)

    def tile(ci, w2_ref, acc):
        lo, hi = _HOFF[ci], _HOFF[ci + 1]
        h = hall[:, lo:hi]
        zn = jax.lax.dot_general(h, w2_ref[...], (((1,), (1,)), ((), ())),
                                 preferred_element_type=jnp.float32)
        r = stats[:, 5 + ci:6 + ci]
        E = jnp.exp2(zn)
        om = 1.0 - r / (1.0 + E)
        contrib = jnp.minimum(jnp.log2(om) * (-_LN2), 100.0)
        csum = contrib[:, 0:128]
        for c in range(1, 7):
            csum = csum + contrib[:, 128 * c:128 * (c + 1)]
        acc[:, 0:128] += csum
        acc[:, 0:_TILE - 896] += contrib[:, 896:_TILE]
        base = ((i - _TOFF[ci]) * _TILE + _COFF[ci]).astype(jnp.float32)
        relf = stats[:, 16:16 + _NL] - base
        inrange = (relf >= 0.0) & (relf <= float(_TILE - 1))
        relc = jnp.clip(relf, 0.0, float(_TILE - 1)).astype(jnp.int32)
        zt = jnp.zeros((_BATCH, _NL), jnp.float32)
        for c in range((_TILE + 127) // 128):
            clo = c * 128
            chi = min(clo + 128, _TILE)
            idx = jnp.clip(relc - clo, 0, chi - clo - 1)
            gth = jnp.take_along_axis(zn[:, clo:chi], idx, axis=1)
            zt = jnp.where((relc >= clo) & (relc < chi), gth, zt)
        pt = r / (1.0 + jnp.exp2(zt))
        corr = jnp.minimum(jnp.log2(pt) * (-_LN2), 100.0) \
            - jnp.minimum(jnp.log2(1.0 - pt) * (-_LN2), 100.0)
        acc[:, 120:120 + _NL] += jnp.where(inrange, corr, 0.0)

    @pl.when(i < _TOFF[1])
    def _t0():
        tile(0, w20_ref, acc0)

    @pl.when((i >= _TOFF[1]) & (i < _TOFF[2]))
    def _t1():
        tile(1, w21_ref, acc1)

    @pl.when(i >= _TOFF[2])
    def _t2():
        tile(2, w22_ref, acc2)

    @pl.when(i == _NT - 1)
    def _fin():
        tail = jnp.zeros((_BATCH, 1), jnp.float32)
        for ci in range(3):
            rm = stats[:, 2 + ci:3 + ci]
            tail = tail + rm * jnp.sum(accs[ci][...], axis=1, keepdims=True)
        total = (stats[:, 0:1] + tail) / stats[:, 1:2]
        out_ref[...] = jnp.full((8, 128), jnp.sum(total) / _BATCH, jnp.float32)


def kernel(input, target, head_W, w1_0, g_0, b_0, w2_0, w1_1, g_1, b_1, w2_1,
           w1_2, g_2, b_2, w2_2):
    f32 = jnp.float32
    hw_pad = jnp.zeros((1024, _IN), f32).at[:head_W.shape[0]].set(head_W)
    w1cat = jnp.zeros((_IN, _IN), f32)
    w1cat = w1cat.at[_HOFF[0]:_HOFF[1]].set(w1_0)
    w1cat = w1cat.at[_HOFF[1]:_HOFF[2]].set(w1_1)
    w1cat = w1cat.at[_HOFF[2]:_HOFF[3]].set(w1_2)
    gb = jnp.zeros((8, _IN), f32)
    gb = gb.at[0, _HOFF[0]:_HOFF[1]].set(g_0).at[1, _HOFF[0]:_HOFF[1]].set(b_0)
    gb = gb.at[0, _HOFF[1]:_HOFF[2]].set(g_1).at[1, _HOFF[1]:_HOFF[2]].set(b_1)
    gb = gb.at[0, _HOFF[2]:_HOFF[3]].set(g_2).at[1, _HOFF[2]:_HOFF[3]].set(b_2)
    targ_pad = jnp.full((_BATCH, 128), -1, jnp.int32).at[:, :_NL].set(target)

    const = lambda i: (0, 0)
    out = pl.pallas_call(
        _body,
        grid=(_NT,),
        in_specs=[
            pl.BlockSpec((_BATCH, _IN), const),
            pl.BlockSpec((1024, _IN), const),
            pl.BlockSpec((_IN, _IN), const),
            pl.BlockSpec((8, _IN), const),
            pl.BlockSpec((_BATCH, 128), const),
            pl.BlockSpec((_TILE, _HSZ[0]), lambda i: (jnp.clip(i, 0, 8), 0)),
            pl.BlockSpec((_TILE, _HSZ[1]), lambda i: (jnp.clip(i - _TOFF[1], 0, 29), 0)),
            pl.BlockSpec((_TILE, _HSZ[2]), lambda i: (jnp.clip(i - _TOFF[2], 0, 59), 0)),
        ],
        out_specs=pl.BlockSpec((8, 128), const),
        out_shape=jax.ShapeDtypeStruct((8, 128), f32),
        scratch_shapes=[
            pltpu.VMEM((_BATCH, 128), f32),
            pltpu.VMEM((_BATCH, _IN), f32),
            pltpu.VMEM((_BATCH, 128), f32),
            pltpu.VMEM((_BATCH, 128), f32),
            pltpu.VMEM((_BATCH, 128), f32),
        ],
    )(input, hw_pad, w1cat, gb, targ_pad, w2_0, w2_1, w2_2)
    return out[0, 0]


# speedup vs baseline: 1.8653x; 1.8653x over previous
import jax
import jax.numpy as jnp
from jax import lax
from jax.experimental import pallas as pl
from jax.experimental.pallas import tpu as pltpu
from jax.experimental.pallas import tpu_sc as plsc

_IN = 128
_BATCH = 1024
_NL = 5
_NLP = 8
_SHORT = 1000
_CUT = [1000, 10000, 40000, 100000]
_CS = [9000, 30000, 60000]
_HSZ = [64, 32, 16]
_HOFF = [0, 64, 96, 112]
_TILE = 1000
_TOFF = [0, 9, 39, 99]
_COFF = [0, 9000, 39000]
_NT = 99
_W2W = 128

_LOG2E = 1.4426950408889634
_LN2 = 0.6931471805599453


def _prep_body(x_ref, hw_ref, w1_ref, gb_ref, targ_ref,
               hall_ref, stats_ref, tdx_ref):
    x = x_ref[...]
    targ = targ_ref[:, 0:_NL]
    ho = lax.dot_general(x, hw_ref[...], (((1,), (1,)), ((), ())),
                         preferred_element_type=jnp.float32)
    hraw = lax.dot_general(x, w1_ref[...], (((1,), (1,)), ((), ())),
                           preferred_element_type=jnp.float32)
    g = gb_ref[0:1, :]
    b = gb_ref[1:2, :]
    for ci in range(3):
        lo, hi = _HOFF[ci], _HOFF[ci + 1]
        hseg = hraw[:, lo:hi]
        mu = jnp.mean(hseg, axis=1, keepdims=True)
        var = jnp.mean((hseg - mu) ** 2, axis=1, keepdims=True)
        hn = (hseg - mu) / jnp.sqrt(var + 1e-5) * g[:, lo:hi] + b[:, lo:hi]
        hall_ref[:, lo:hi] = jnp.maximum(hn, 0.0) * (-_LOG2E)
    hall_ref[:, _HOFF[3]:_IN] = jnp.zeros((_BATCH, _IN - _HOFF[3]), jnp.float32)

    colid = lax.broadcasted_iota(jnp.int32, (_BATCH, 1024), 1)
    mh = jnp.zeros((_BATCH, 1024), jnp.bool_)
    for k in range(_NL):
        mh = mh | (targ[:, k:k + 1] == colid)
    mh = mh & (colid < _SHORT)
    sp = jnp.maximum(ho, 0.0) + jnp.log1p(jnp.exp(-jnp.abs(ho)))
    maskH = (colid < _SHORT).astype(jnp.float32)
    head_loss = jnp.sum(sp * maskH, axis=1, keepdims=True) \
        - jnp.sum(jnp.where(mh, ho, 0.0), axis=1, keepdims=True)

    num = jnp.full((_BATCH, 1), float(_SHORT), jnp.float32)
    for ci in range(3):
        lo, hi = _CUT[ci], _CUT[ci + 1]
        rm = jnp.zeros((_BATCH, 1), jnp.bool_)
        for k in range(_NL):
            tk = targ[:, k:k + 1]
            rm = rm | ((tk >= lo) & (tk < hi))
        rmf = rm.astype(jnp.float32)
        logit = ho[:, _SHORT + ci:_SHORT + ci + 1]
        sp_r = jnp.maximum(logit, 0.0) + jnp.log1p(jnp.exp(-jnp.abs(logit)))
        head_loss = head_loss + (1.0 - rmf) * sp_r
        num = num + jnp.where(rm, float(_CS[ci]), 1.0)
        stats_ref[:, 2 + ci:3 + ci] = rmf
        stats_ref[:, 5 + ci:6 + ci] = ho[:, _SHORT + ci:_SHORT + ci + 1]
    stats_ref[:, 0:1] = head_loss
    stats_ref[:, 1:2] = num

    tdx_ref[...] = jnp.zeros((_BATCH, 128), jnp.int32)
    for k in range(_NL):
        tk = targ[:, k:k + 1]
        dup = jnp.zeros((_BATCH, 1), jnp.bool_)
        for j in range(k):
            dup = dup | (targ[:, j:j + 1] == tk)
        tcol = jnp.where(dup, -1, tk - _SHORT)
        stats_ref[:, 16 + k:17 + k] = tcol.astype(jnp.float32)
        tdx_ref[:, k:k + 1] = jnp.maximum(tcol, 0)


def _dense_body(hall_ref, stats_ref, w20_ref, w21_ref, w22_ref,
                tails_ref, acc0, acc1, acc2):
    i = pl.program_id(0)
    accs = [acc0, acc1, acc2]

    @pl.when(i == 0)
    def _init():
        for acc in accs:
            acc[...] = jnp.zeros((_BATCH, 128), jnp.float32)

    def tile(ci, w2_ref, acc):
        lo, hi = _HOFF[ci], _HOFF[ci + 1]
        zn = lax.dot_general(hall_ref[:, lo:hi], w2_ref[...],
                             (((1,), (1,)), ((), ())),
                             preferred_element_type=jnp.float32)
        r = jax.nn.sigmoid(stats_ref[:, 5 + ci:6 + ci])
        d = 1.0 + jnp.exp2(jnp.minimum(zn, 126.0))
        contrib = jnp.minimum((jnp.log2(d) - jnp.log2(d - r)) * _LN2, 100.0)
        csum = contrib[:, 0:128]
        for c in range(1, 7):
            csum = csum + contrib[:, 128 * c:128 * (c + 1)]
        acc[:, 0:128] += csum
        acc[:, 0:_TILE - 896] += contrib[:, 896:_TILE]

    @pl.when(i < _TOFF[1])
    def _t0():
        tile(0, w20_ref, acc0)

    @pl.when((i >= _TOFF[1]) & (i < _TOFF[2]))
    def _t1():
        tile(1, w21_ref, acc1)

    @pl.when(i >= _TOFF[2])
    def _t2():
        tile(2, w22_ref, acc2)

    @pl.when(i == _NT - 1)
    def _fin():
        for ci in range(3):
            tails_ref[:, ci:ci + 1] = jnp.sum(accs[ci][...], axis=1,
                                              keepdims=True)
        tails_ref[:, 3:128] = jnp.zeros((_BATCH, 125), jnp.float32)


_NC, _NS, _L = 2, 16, 16
_NW = _NC * _NS
_RPW = _BATCH // _NW
_PPW = _RPW * _NLP
_NGRP = _PPW // _L


def _sc_body(hall_hbm, tdx2_hbm, w2p_hbm, znt_hbm,
             hall_v, idx_v0, idx_v1, w2a, w2b, out_v, sem):
    wid = lax.axis_index("s") * _NC + lax.axis_index("c")
    rbase = wid * _RPW
    pltpu.sync_copy(hall_hbm.at[pl.ds(rbase, _RPW)], hall_v)
    pltpu.sync_copy(tdx2_hbm.at[pl.ds(wid * _PPW, 128)], idx_v0)
    pltpu.sync_copy(tdx2_hbm.at[pl.ds(wid * _PPW + 128, 128)], idx_v1)
    cp0 = pltpu.async_copy(w2p_hbm.at[idx_v0], w2a, sem)
    cp1 = pltpu.async_copy(w2p_hbm.at[idx_v1], w2b, sem)
    cp0.wait()
    cp1.wait()

    def half_loop(half, w2_ref):
        def group(g, _):
            for j in range(_L):
                row = g * _L + j
                b = (half * 128 + row) // _NLP
                acc = jnp.zeros((_L,), jnp.float32)
                for c in range(_IN // _L):
                    acc = acc + w2_ref[row, pl.ds(c * _L, _L)] \
                        * hall_v[b, pl.ds(c * _L, _L)]
                out_v[half * 128 + row, :] = acc
            return 0
        lax.fori_loop(0, 128 // _L, group, 0)

    half_loop(0, w2a)
    half_loop(1, w2b)
    pltpu.sync_copy(out_v, znt_hbm.at[pl.ds(wid * _PPW, _PPW)])


def _combine_body(stats_ref, tails_ref, znt_ref, out_ref):
    tcol = stats_ref[:, 16:16 + _NL]
    valid = tcol >= 0.0
    total = jnp.zeros((_BATCH, 1), jnp.float32)
    rsig = [jax.nn.sigmoid(stats_ref[:, 5 + ci:6 + ci]) for ci in range(3)]
    for ci in range(3):
        rm = stats_ref[:, 2 + ci:3 + ci]
        total = total + rm * tails_ref[:, ci:ci + 1]
    rpair = jnp.where(tcol >= float(_COFF[2]), rsig[2],
                      jnp.where(tcol >= float(_COFF[1]), rsig[1], rsig[0]))
    znt = jnp.concatenate(
        [jnp.sum(znt_ref[:, 16 * k:16 * (k + 1)], axis=1, keepdims=True)
         for k in range(_NL)], axis=1)
    et = jnp.exp2(jnp.minimum(znt, 126.0))
    pt = rpair / (1.0 + et)
    corr = jnp.minimum(jnp.log2(pt) * (-_LN2), 100.0) \
        - jnp.minimum(jnp.log2(1.0 - pt) * (-_LN2), 100.0)
    total = total + jnp.sum(jnp.where(valid, corr, 0.0), axis=1, keepdims=True)
    per_row = (stats_ref[:, 0:1] + total) / stats_ref[:, 1:2]
    out_ref[...] = jnp.full((8, 128), jnp.sum(per_row) / _BATCH, jnp.float32)


def kernel(input, target, head_W, w1_0, g_0, b_0, w2_0, w1_1, g_1, b_1, w2_1,
           w1_2, g_2, b_2, w2_2):
    f32 = jnp.float32
    hw_pad = jnp.zeros((1024, _IN), f32).at[:head_W.shape[0]].set(head_W)
    w1cat = jnp.zeros((_IN, _IN), f32)
    w1cat = w1cat.at[_HOFF[0]:_HOFF[1]].set(w1_0)
    w1cat = w1cat.at[_HOFF[1]:_HOFF[2]].set(w1_1)
    w1cat = w1cat.at[_HOFF[2]:_HOFF[3]].set(w1_2)
    gb = jnp.zeros((8, _IN), f32)
    gb = gb.at[0, _HOFF[0]:_HOFF[1]].set(g_0).at[1, _HOFF[0]:_HOFF[1]].set(b_0)
    gb = gb.at[0, _HOFF[1]:_HOFF[2]].set(g_1).at[1, _HOFF[1]:_HOFF[2]].set(b_1)
    gb = gb.at[0, _HOFF[2]:_HOFF[3]].set(g_2).at[1, _HOFF[2]:_HOFF[3]].set(b_2)
    targ_pad = jnp.full((_BATCH, 128), -1, jnp.int32).at[:, :_NL].set(target)
    w2pack = jnp.concatenate([
        jnp.pad(w2_0, ((0, 0), (_HOFF[0], _W2W - _HOFF[1]))),
        jnp.pad(w2_1, ((0, 0), (_HOFF[1], _W2W - _HOFF[2]))),
        jnp.pad(w2_2, ((0, 0), (_HOFF[2], _W2W - _HOFF[3]))),
    ], axis=0)

    const = lambda i: (0, 0)

    hall, stats, tdx = pl.pallas_call(
        _prep_body,
        in_specs=[pl.BlockSpec((_BATCH, _IN), None),
                  pl.BlockSpec((1024, _IN), None),
                  pl.BlockSpec((_IN, _IN), None),
                  pl.BlockSpec((8, _IN), None),
                  pl.BlockSpec((_BATCH, 128), None)],
        out_specs=[pl.BlockSpec((_BATCH, _IN), None),
                   pl.BlockSpec((_BATCH, 128), None),
                   pl.BlockSpec((_BATCH, 128), None)],
        out_shape=[jax.ShapeDtypeStruct((_BATCH, _IN), f32),
                   jax.ShapeDtypeStruct((_BATCH, 128), f32),
                   jax.ShapeDtypeStruct((_BATCH, 128), jnp.int32)],
    )(input, hw_pad, w1cat, gb, targ_pad)

    tails = pl.pallas_call(
        _dense_body,
        grid=(_NT,),
        in_specs=[
            pl.BlockSpec((_BATCH, _IN), const),
            pl.BlockSpec((_BATCH, 128), const),
            pl.BlockSpec((_TILE, _HSZ[0]), lambda i: (jnp.clip(i, 0, 8), 0)),
            pl.BlockSpec((_TILE, _HSZ[1]),
                         lambda i: (jnp.clip(i - _TOFF[1], 0, 29), 0)),
            pl.BlockSpec((_TILE, _HSZ[2]),
                         lambda i: (jnp.clip(i - _TOFF[2], 0, 59), 0)),
        ],
        out_specs=pl.BlockSpec((_BATCH, 128), const),
        out_shape=jax.ShapeDtypeStruct((_BATCH, 128), f32),
        scratch_shapes=[pltpu.VMEM((_BATCH, 128), f32)] * 3,
    )(hall, stats, w2_0, w2_1, w2_2)

    tdx2 = tdx[:, 0:_NLP].reshape(-1)
    znt_flat = pl.kernel(
        _sc_body,
        out_type=jax.ShapeDtypeStruct((_BATCH * _NLP, _L), f32),
        mesh=plsc.VectorSubcoreMesh(core_axis_name="c", subcore_axis_name="s"),
        scratch_types=[
            pltpu.VMEM((_RPW, _IN), f32),
            pltpu.VMEM((128,), jnp.int32),
            pltpu.VMEM((128,), jnp.int32),
            pltpu.VMEM((128, _W2W), f32),
            pltpu.VMEM((128, _W2W), f32),
            pltpu.VMEM((_PPW, _L), f32),
            pltpu.SemaphoreType.DMA,
        ],
    )(hall, tdx2, w2pack)
    znt = znt_flat.reshape(_BATCH, _NLP * _L)

    out = pl.pallas_call(
        _combine_body,
        in_specs=[pl.BlockSpec((_BATCH, 128), None)] * 3,
        out_specs=pl.BlockSpec((8, 128), None),
        out_shape=jax.ShapeDtypeStruct((8, 128), f32),
    )(stats, tails, znt)
    return out[0, 0]


# speedup vs baseline: 1.9305x; 1.0350x over previous
import jax
import jax.numpy as jnp
from jax import lax
from jax.experimental import pallas as pl
from jax.experimental.pallas import tpu as pltpu
from jax.experimental.pallas import tpu_sc as plsc

_IN = 128
_BATCH = 1024
_NL = 5
_NLP = 8
_SHORT = 1000
_CUT = [1000, 10000, 40000, 100000]
_CS = [9000, 30000, 60000]
_HSZ = [64, 32, 16]
_HOFF = [0, 64, 96, 112]
_TILE = 1000
_TOFF = [0, 9, 39, 99]
_COFF = [0, 9000, 39000]
_NT = 99
_W2W = 128

_LOG2E = 1.4426950408889634
_LN2 = 0.6931471805599453


def _prep_body(x_ref, hw_ref, w1_ref, gb_ref, targ_ref,
               hall_ref, stats_ref, tdx_ref):
    x = x_ref[...]
    targ = targ_ref[:, 0:_NL]
    ho = lax.dot_general(x, hw_ref[...], (((1,), (1,)), ((), ())),
                         preferred_element_type=jnp.float32)
    hraw = lax.dot_general(x, w1_ref[...], (((1,), (1,)), ((), ())),
                           preferred_element_type=jnp.float32)
    g = gb_ref[0:1, :]
    b = gb_ref[1:2, :]
    for ci in range(3):
        lo, hi = _HOFF[ci], _HOFF[ci + 1]
        hseg = hraw[:, lo:hi]
        mu = jnp.mean(hseg, axis=1, keepdims=True)
        var = jnp.mean((hseg - mu) ** 2, axis=1, keepdims=True)
        hn = (hseg - mu) / jnp.sqrt(var + 1e-5) * g[:, lo:hi] + b[:, lo:hi]
        hall_ref[:, lo:hi] = jnp.maximum(hn, 0.0) * (-_LOG2E)
    hall_ref[:, _HOFF[3]:_IN] = jnp.zeros((_BATCH, _IN - _HOFF[3]), jnp.float32)

    colid = lax.broadcasted_iota(jnp.int32, (_BATCH, 1024), 1)
    mh = jnp.zeros((_BATCH, 1024), jnp.bool_)
    for k in range(_NL):
        mh = mh | (targ[:, k:k + 1] == colid)
    mh = mh & (colid < _SHORT)
    sp = jnp.maximum(ho, 0.0) + jnp.log1p(jnp.exp(-jnp.abs(ho)))
    maskH = (colid < _SHORT).astype(jnp.float32)
    head_loss = jnp.sum(sp * maskH, axis=1, keepdims=True) \
        - jnp.sum(jnp.where(mh, ho, 0.0), axis=1, keepdims=True)

    num = jnp.full((_BATCH, 1), float(_SHORT), jnp.float32)
    for ci in range(3):
        lo, hi = _CUT[ci], _CUT[ci + 1]
        rm = jnp.zeros((_BATCH, 1), jnp.bool_)
        for k in range(_NL):
            tk = targ[:, k:k + 1]
            rm = rm | ((tk >= lo) & (tk < hi))
        rmf = rm.astype(jnp.float32)
        logit = ho[:, _SHORT + ci:_SHORT + ci + 1]
        sp_r = jnp.maximum(logit, 0.0) + jnp.log1p(jnp.exp(-jnp.abs(logit)))
        head_loss = head_loss + (1.0 - rmf) * sp_r
        num = num + jnp.where(rm, float(_CS[ci]), 1.0)
        stats_ref[:, 2 + ci:3 + ci] = rmf
        stats_ref[:, 5 + ci:6 + ci] = jax.nn.sigmoid(
            ho[:, _SHORT + ci:_SHORT + ci + 1])
    stats_ref[:, 0:1] = head_loss
    stats_ref[:, 1:2] = num

    tdx_ref[...] = jnp.zeros((_BATCH, 128), jnp.int32)
    for k in range(_NL):
        tk = targ[:, k:k + 1]
        dup = jnp.zeros((_BATCH, 1), jnp.bool_)
        for j in range(k):
            dup = dup | (targ[:, j:j + 1] == tk)
        tcol = jnp.where(dup, -1, tk - _SHORT)
        stats_ref[:, 16 + k:17 + k] = tcol.astype(jnp.float32)
        tdx_ref[:, k:k + 1] = jnp.maximum(tcol, 0)


def _dense_body(hall_ref, stats_ref, w20_ref, w21_ref, w22_ref,
                tails_ref, acc0, acc1, acc2):
    i = pl.program_id(0)
    accs = [acc0, acc1, acc2]

    @pl.when(i == 0)
    def _init():
        for acc in accs:
            acc[...] = jnp.zeros((_BATCH, 128), jnp.float32)

    def tile(ci, w2_ref, acc):
        lo, hi = _HOFF[ci], _HOFF[ci + 1]
        zn = lax.dot_general(hall_ref[:, lo:hi], w2_ref[...],
                             (((1,), (1,)), ((), ())),
                             preferred_element_type=jnp.float32)
        r = stats_ref[:, 5 + ci:6 + ci]
        d = 1.0 + jnp.exp2(jnp.minimum(zn, 126.0))
        contrib = jnp.minimum(jnp.log2(d) - jnp.log2(d - r), 100.0 / _LN2)
        csum = contrib[:, 0:128]
        for c in range(1, 7):
            csum = csum + contrib[:, 128 * c:128 * (c + 1)]
        acc[:, 0:128] += csum
        acc[:, 0:_TILE - 896] += contrib[:, 896:_TILE]

    @pl.when(i < _TOFF[1])
    def _t0():
        tile(0, w20_ref, acc0)

    @pl.when((i >= _TOFF[1]) & (i < _TOFF[2]))
    def _t1():
        tile(1, w21_ref, acc1)

    @pl.when(i >= _TOFF[2])
    def _t2():
        tile(2, w22_ref, acc2)

    @pl.when(i == _NT - 1)
    def _fin():
        for ci in range(3):
            tails_ref[:, ci:ci + 1] = jnp.sum(accs[ci][...], axis=1,
                                              keepdims=True) * _LN2


_NC, _NS, _L = 2, 16, 16
_NW = _NC * _NS
_RPW = _BATCH // _NW
_PPW = _RPW * _NLP
_NGRP = _PPW // _L


def _sc_body(tdx2_hbm, w2p_hbm, wg_hbm, idx_v0, idx_v1, w2a, w2b, sem):
    wid = lax.axis_index("s") * _NC + lax.axis_index("c")
    pltpu.sync_copy(tdx2_hbm.at[pl.ds(wid * _PPW, 128)], idx_v0)
    pltpu.sync_copy(tdx2_hbm.at[pl.ds(wid * _PPW + 128, 128)], idx_v1)
    cp0 = pltpu.async_copy(w2p_hbm.at[idx_v0], w2a, sem)
    cp1 = pltpu.async_copy(w2p_hbm.at[idx_v1], w2b, sem)
    cp0.wait()
    cp1.wait()
    pltpu.sync_copy(w2a, wg_hbm.at[pl.ds(wid * _PPW, 128)])
    pltpu.sync_copy(w2b, wg_hbm.at[pl.ds(wid * _PPW + 128, 128)])


def _combine_body(stats_ref, tails_ref, hall_ref, wg0_ref, wg1_ref, wg2_ref,
                  wg3_ref, wg4_ref, out_ref):
    tcol = stats_ref[:, 16:16 + _NL]
    valid = tcol >= 0.0
    total = jnp.zeros((_BATCH, 1), jnp.float32)
    rsig = [stats_ref[:, 5 + ci:6 + ci] for ci in range(3)]
    for ci in range(3):
        rm = stats_ref[:, 2 + ci:3 + ci]
        total = total + rm * tails_ref[:, ci:ci + 1]
    rpair = jnp.where(tcol >= float(_COFF[2]), rsig[2],
                      jnp.where(tcol >= float(_COFF[1]), rsig[1], rsig[0]))
    h = hall_ref[...]
    znt = jnp.concatenate(
        [jnp.sum(wg_ref[...] * h, axis=1, keepdims=True)
         for wg_ref in (wg0_ref, wg1_ref, wg2_ref, wg3_ref, wg4_ref)],
        axis=1)
    et = jnp.exp2(jnp.minimum(znt, 126.0))
    pt = rpair / (1.0 + et)
    corr = jnp.minimum(jnp.log2(pt) * (-_LN2), 100.0) \
        - jnp.minimum(jnp.log2(1.0 - pt) * (-_LN2), 100.0)
    total = total + jnp.sum(jnp.where(valid, corr, 0.0), axis=1, keepdims=True)
    per_row = (stats_ref[:, 0:1] + total) / stats_ref[:, 1:2]
    out_ref[...] = jnp.full((8, 128), jnp.sum(per_row) / _BATCH, jnp.float32)


def kernel(input, target, head_W, w1_0, g_0, b_0, w2_0, w1_1, g_1, b_1, w2_1,
           w1_2, g_2, b_2, w2_2):
    f32 = jnp.float32
    hw_pad = jnp.zeros((1024, _IN), f32).at[:head_W.shape[0]].set(head_W)
    w1cat = jnp.zeros((_IN, _IN), f32)
    w1cat = w1cat.at[_HOFF[0]:_HOFF[1]].set(w1_0)
    w1cat = w1cat.at[_HOFF[1]:_HOFF[2]].set(w1_1)
    w1cat = w1cat.at[_HOFF[2]:_HOFF[3]].set(w1_2)
    gb = jnp.zeros((8, _IN), f32)
    gb = gb.at[0, _HOFF[0]:_HOFF[1]].set(g_0).at[1, _HOFF[0]:_HOFF[1]].set(b_0)
    gb = gb.at[0, _HOFF[1]:_HOFF[2]].set(g_1).at[1, _HOFF[1]:_HOFF[2]].set(b_1)
    gb = gb.at[0, _HOFF[2]:_HOFF[3]].set(g_2).at[1, _HOFF[2]:_HOFF[3]].set(b_2)
    targ_pad = jnp.full((_BATCH, 128), -1, jnp.int32).at[:, :_NL].set(target)
    w2pack = jnp.concatenate([
        jnp.pad(w2_0, ((0, 0), (_HOFF[0], _W2W - _HOFF[1]))),
        jnp.pad(w2_1, ((0, 0), (_HOFF[1], _W2W - _HOFF[2]))),
        jnp.pad(w2_2, ((0, 0), (_HOFF[2], _W2W - _HOFF[3]))),
    ], axis=0)

    const = lambda i: (0, 0)

    hall, stats, tdx = pl.pallas_call(
        _prep_body,
        in_specs=[pl.BlockSpec((_BATCH, _IN), None),
                  pl.BlockSpec((1024, _IN), None),
                  pl.BlockSpec((_IN, _IN), None),
                  pl.BlockSpec((8, _IN), None),
                  pl.BlockSpec((_BATCH, 128), None)],
        out_specs=[pl.BlockSpec((_BATCH, _IN), None),
                   pl.BlockSpec((_BATCH, 128), None),
                   pl.BlockSpec((_BATCH, 128), None)],
        out_shape=[jax.ShapeDtypeStruct((_BATCH, _IN), f32),
                   jax.ShapeDtypeStruct((_BATCH, 128), f32),
                   jax.ShapeDtypeStruct((_BATCH, 128), jnp.int32)],
    )(input, hw_pad, w1cat, gb, targ_pad)

    tails = pl.pallas_call(
        _dense_body,
        grid=(_NT,),
        in_specs=[
            pl.BlockSpec((_BATCH, _IN), const),
            pl.BlockSpec((_BATCH, 128), const),
            pl.BlockSpec((_TILE, _HSZ[0]), lambda i: (jnp.clip(i, 0, 8), 0)),
            pl.BlockSpec((_TILE, _HSZ[1]),
                         lambda i: (jnp.clip(i - _TOFF[1], 0, 29), 0)),
            pl.BlockSpec((_TILE, _HSZ[2]),
                         lambda i: (jnp.clip(i - _TOFF[2], 0, 59), 0)),
        ],
        out_specs=pl.BlockSpec((_BATCH, 128), const),
        out_shape=jax.ShapeDtypeStruct((_BATCH, 128), f32),
        scratch_shapes=[pltpu.VMEM((_BATCH, 128), f32)] * 3,
    )(hall, stats, w2_0, w2_1, w2_2)

    tdx2 = tdx[:, 0:_NLP].reshape(-1)
    wg = pl.kernel(
        _sc_body,
        out_type=jax.ShapeDtypeStruct((_BATCH * _NLP, _W2W), f32),
        mesh=plsc.VectorSubcoreMesh(core_axis_name="c", subcore_axis_name="s"),
        scratch_types=[
            pltpu.VMEM((128,), jnp.int32),
            pltpu.VMEM((128,), jnp.int32),
            pltpu.VMEM((128, _W2W), f32),
            pltpu.VMEM((128, _W2W), f32),
            pltpu.SemaphoreType.DMA,
        ],
    )(tdx2, w2pack)
    wgr = wg.reshape(_BATCH, _NLP, _W2W)
    wgs = [wgr[:, k, :] for k in range(_NL)]

    out = pl.pallas_call(
        _combine_body,
        in_specs=[pl.BlockSpec((_BATCH, 128), None)] * (3 + _NL),
        out_specs=pl.BlockSpec((8, 128), None),
        out_shape=jax.ShapeDtypeStruct((8, 128), f32),
    )(stats, tails, hall, *wgs)
    return out[0, 0]
